# baseline (speedup 1.0000x reference)
import functools

import jax
import jax.numpy as jnp
import numpy as np
from jax import lax
from jax.experimental import pallas as pl
from jax.experimental.pallas import tpu as pltpu
from jax.experimental.pallas import tpu_sc as plsc

N = 10000
D = 128
E = 320000
B = 64
EPS = 1e-5

NP = 10240
EP = 327680
NC = 2
NS = 16
NW = NC * NS
ET = EP // NW
EC = 64
CH = ET // EC
NG = 4
GRP = CH // NG
RPT = NP // NS

RB = 512
NB = NP // RB


@functools.cache
def _build_sc_degree():
    mesh = plsc.VectorSubcoreMesh(core_axis_name="c", subcore_axis_name="s")
    return functools.partial(
        pl.kernel,
        mesh=mesh,
        out_type=jax.ShapeDtypeStruct((NC * NP,), jnp.float32),
        scratch_types=[
            pltpu.VMEM((CH, EC), jnp.int32),
            pltpu.VMEM((EC,), jnp.float32),
            pltpu.VMEM_SHARED((NP,), jnp.float32),
            pltpu.SemaphoreType.DMA,
        ],
    )(_sc_degree_body)


def _sc_degree_body(dst_hbm, ones_hbm, z1_hbm, out_hbm, idx_v, ones_v, deg_sh,
                    sem):
    c = lax.axis_index("c")
    s = lax.axis_index("s")
    wid = c * NS + s
    pltpu.sync_copy(dst_hbm.at[wid], idx_v)
    pltpu.sync_copy(ones_hbm, ones_v)
    pltpu.sync_copy(z1_hbm, deg_sh.at[pl.ds(s * RPT, RPT)])
    plsc.subcore_barrier()

    def fire(j, _):
        pltpu.async_copy(ones_v, deg_sh.at[idx_v.at[j]], sem, add=True)
        return _

    def drain(j, _):
        pltpu.make_async_copy(ones_v, deg_sh.at[idx_v.at[j]], sem).wait()
        return _

    lax.fori_loop(0, CH, fire, None)
    lax.fori_loop(0, CH, drain, None)
    plsc.subcore_barrier()
    pltpu.sync_copy(deg_sh.at[pl.ds(s * RPT, RPT)],
                    out_hbm.at[pl.ds(c * NP + s * RPT, RPT)])


def _sc_degree(dstp, ones128, z1):
    return _build_sc_degree()(dstp, ones128, z1)


@functools.cache
def _build_sc_scatter():
    mesh = plsc.VectorSubcoreMesh(core_axis_name="c", subcore_axis_name="s")
    return functools.partial(
        pl.kernel,
        mesh=mesh,
        out_type=jax.ShapeDtypeStruct((NC * NP, D), jnp.float32),
        scratch_types=[
            pltpu.VMEM((GRP, EC), jnp.int32),
            pltpu.VMEM((GRP, EC), jnp.int32),
            pltpu.VMEM((EC, D), jnp.float32),
            pltpu.VMEM((EC, D), jnp.float32),
            pltpu.VMEM((EC, D), jnp.float32),
            pltpu.VMEM((EC, D), jnp.float32),
            pltpu.VMEM_SHARED((NP, D), jnp.float32),
            pltpu.SemaphoreType.DMA,
            pltpu.SemaphoreType.DMA,
            pltpu.SemaphoreType.DMA,
            pltpu.SemaphoreType.DMA,
            pltpu.SemaphoreType.DMA,
            pltpu.SemaphoreType.DMA,
            pltpu.SemaphoreType.DMA,
            pltpu.SemaphoreType.DMA,
        ],
    )(_sc_scatter_body)


def _sc_scatter_body(u_hbm, src_hbm, dst_hbm, zrows_hbm, out_hbm,
                     src_v, dst_v, rows0, rows1, rows2, rows3, acc_sh,
                     gsem0, gsem1, gsem2, gsem3, ssem0, ssem1, ssem2, ssem3):
    c = lax.axis_index("c")
    s = lax.axis_index("s")
    wid = c * NS + s
    cz = pltpu.async_copy(zrows_hbm, acc_sh.at[pl.ds(s * RPT, RPT)], ssem0)
    c0 = pltpu.async_copy(src_hbm.at[wid * NG], src_v, gsem0)
    c1 = pltpu.async_copy(dst_hbm.at[wid * NG], dst_v, gsem1)
    c0.wait()
    c1.wait()
    cz.wait()
    plsc.subcore_barrier()

    rows = (rows0, rows1, rows2, rows3)
    gsem = (gsem0, gsem1, gsem2, gsem3)
    ssem = (ssem0, ssem1, ssem2, ssem3)

    def body(j, _):
        for b in range(4):
            @pl.when(j % 4 == b)
            def _branch():
                pb = (b + 2) % 4
                @pl.when(j >= 2)
                def _drain_prev():
                    pltpu.make_async_copy(
                        rows[pb], acc_sh.at[dst_v.at[j - 2]], ssem[pb]).wait()

                @pl.when(j + 2 < GRP)
                def _prefetch():
                    pltpu.async_copy(u_hbm.at[src_v.at[j + 2]], rows[pb],
                                     gsem[pb])

                pltpu.make_async_copy(u_hbm.at[src_v.at[j]], rows[b],
                                      gsem[b]).wait()
                pltpu.async_copy(rows[b], acc_sh.at[dst_v.at[j]], ssem[b],
                                 add=True)
        return _

    for g in range(NG):
        if g > 0:
            cg0 = pltpu.async_copy(src_hbm.at[wid * NG + g], src_v, gsem0)
            cg1 = pltpu.async_copy(dst_hbm.at[wid * NG + g], dst_v, gsem1)
            cg0.wait()
            cg1.wait()
        pltpu.async_copy(u_hbm.at[src_v.at[0]], rows0, gsem0)
        pltpu.async_copy(u_hbm.at[src_v.at[1]], rows1, gsem1)
        lax.fori_loop(0, GRP, body, None)
        pltpu.make_async_copy(rows[(GRP - 2) % 4], acc_sh.at[dst_v.at[GRP - 2]],
                              ssem[(GRP - 2) % 4]).wait()
        pltpu.make_async_copy(rows[(GRP - 1) % 4], acc_sh.at[dst_v.at[GRP - 1]],
                              ssem[(GRP - 1) % 4]).wait()
    plsc.subcore_barrier()
    pltpu.sync_copy(acc_sh.at[pl.ds(s * RPT, RPT)],
                    out_hbm.at[pl.ds(c * NP + s * RPT, RPT)])


def _sc_scatter(u, srcp, dstp, zrows):
    return _build_sc_scatter()(u, srcp, dstp, zrows)


def _dinv_block(d0_ref, d1_ref):
    return lax.rsqrt(d0_ref[0, 0, :] + d1_ref[0, 0, :] + 1.0)


def _ka_body(x_ref, w_ref, d0_ref, d1_ref, u_ref):
    dinv = _dinv_block(d0_ref, d1_ref)
    hw = jnp.dot(x_ref[...], w_ref[...], preferred_element_type=jnp.float32)
    u_ref[...] = hw * dinv[:, None]


def _tc_layer0(xp, W1, deg0, deg1):
    return pl.pallas_call(
        _ka_body,
        grid=(NB,),
        in_specs=[
            pl.BlockSpec((RB, D), lambda i: (i, 0)),
            pl.BlockSpec((D, D), lambda i: (0, 0)),
            pl.BlockSpec((1, 1, RB), lambda i: (i, 0, 0)),
            pl.BlockSpec((1, 1, RB), lambda i: (i, 0, 0)),
        ],
        out_specs=pl.BlockSpec((RB, D), lambda i: (i, 0)),
        out_shape=jax.ShapeDtypeStruct((NP, D), jnp.float32),
    )(xp, W1, deg0, deg1)


def _layer_post(acc_a, acc_b, u, dinv, b, g, be, res):
    a = (acc_a + acc_b + u) * dinv[:, None] + b[0]
    m = jnp.mean(a, axis=1, keepdims=True)
    v = jnp.mean((a - m) ** 2, axis=1, keepdims=True)
    ln = (a - m) * lax.rsqrt(v + EPS) * g[0] + be[0]
    return jnp.maximum(ln, 0.0) + res


def _kb_body(aa_ref, ab_ref, u1_ref, d0_ref, d1_ref, b1_ref, g1_ref, be1_ref,
             x_ref, w2_ref, h_ref, u2_ref):
    dinv = _dinv_block(d0_ref, d1_ref)
    h = _layer_post(aa_ref[...], ab_ref[...], u1_ref[...], dinv,
                    b1_ref[...], g1_ref[...], be1_ref[...], x_ref[...])
    h_ref[...] = h
    hw2 = jnp.dot(h, w2_ref[...], preferred_element_type=jnp.float32)
    u2_ref[...] = hw2 * dinv[:, None]


def _tc_layer1(acc1, u1, deg0, deg1, b1, g1, be1, xp, W2):
    row = pl.BlockSpec((RB, D), lambda i: (i, 0))
    row_hi = pl.BlockSpec((RB, D), lambda i: (i + NB, 0))
    vec = pl.BlockSpec((1, D), lambda i: (0, 0))
    dspec = pl.BlockSpec((1, 1, RB), lambda i: (i, 0, 0))
    return pl.pallas_call(
        _kb_body,
        grid=(NB,),
        in_specs=[row, row_hi, row, dspec, dspec, vec, vec, vec, row,
                  pl.BlockSpec((D, D), lambda i: (0, 0))],
        out_specs=[row, row],
        out_shape=[jax.ShapeDtypeStruct((NP, D), jnp.float32),
                   jax.ShapeDtypeStruct((NP, D), jnp.float32)],
    )(acc1, acc1, u1, deg0, deg1, b1, g1, be1, xp, W2)


def _kc_body(aa_ref, ab_ref, u2_ref, d0_ref, d1_ref, b2_ref, g2_ref, be2_ref,
             h_ref, ids_ref, wo_ref, bo_ref, out_ref, sums_ref, cnts_ref):
    dinv = _dinv_block(d0_ref, d1_ref)
    t = _layer_post(aa_ref[...], ab_ref[...], u2_ref[...], dinv,
                    b2_ref[...], g2_ref[...], be2_ref[...], h_ref[...])
    ids = ids_ref[0, 0, :]
    onehot = (ids[:, None] == lax.broadcasted_iota(jnp.int32, (RB, B), 1))
    onehot = onehot.astype(jnp.float32)

    @pl.when(pl.program_id(0) == 0)
    def _init():
        sums_ref[...] = jnp.zeros_like(sums_ref)
        cnts_ref[...] = jnp.zeros_like(cnts_ref)

    sums_ref[...] += lax.dot_general(
        onehot, t, (((0,), (0,)), ((), ())),
        preferred_element_type=jnp.float32)
    cnts_ref[...] += jnp.broadcast_to(
        jnp.sum(onehot, axis=0)[:, None], (B, D))

    @pl.when(pl.program_id(0) == NB - 1)
    def _project():
        hg = sums_ref[...] / jnp.maximum(cnts_ref[...], 1.0)
        out_ref[...] = jnp.dot(hg, wo_ref[...],
                               preferred_element_type=jnp.float32) + bo_ref[...]


def _tc_layer2_pool(acc2, u2, deg0, deg1, b2, g2, be2, h, batch_r, Wo, bo):
    row = pl.BlockSpec((RB, D), lambda i: (i, 0))
    row_hi = pl.BlockSpec((RB, D), lambda i: (i + NB, 0))
    vec = pl.BlockSpec((1, D), lambda i: (0, 0))
    dspec = pl.BlockSpec((1, 1, RB), lambda i: (i, 0, 0))
    pool = pl.BlockSpec((B, D), lambda i: (0, 0))
    return pl.pallas_call(
        _kc_body,
        grid=(NB,),
        in_specs=[row, row_hi, row, dspec, dspec, vec, vec, vec, row, dspec,
                  pl.BlockSpec((D, D), lambda i: (0, 0)), vec],
        out_specs=pool,
        out_shape=jax.ShapeDtypeStruct((B, D), jnp.float32),
        scratch_shapes=[pltpu.VMEM((B, D), jnp.float32),
                        pltpu.VMEM((B, D), jnp.float32)],
    )(acc2, acc2, u2, deg0, deg1, b2, g2, be2, h, batch_r, Wo,
      bo.reshape(1, D))


def kernel(x, edge_index, batch, W1, b1, g1, be1, W2, b2, g2, be2, Wo, bo):
    src, dst = edge_index[0], edge_index[1]
    pad_ids = jnp.asarray(N + np.arange(EP - E, dtype=np.int32) % (NP - N))
    srcp = jnp.concatenate([src, pad_ids]).reshape(NW * NG, GRP, EC)
    dstp = jnp.concatenate([dst, pad_ids]).reshape(NW * NG, GRP, EC)
    dst_deg = dstp.reshape(NW, CH, EC)
    xp = jnp.pad(x, ((0, NP - N), (0, 0)))
    batch_r = jnp.pad(batch, (0, NP - N), constant_values=B).reshape(NB, 1, RB)

    ones_ec = jnp.ones((EC,), jnp.float32)
    z1 = jnp.zeros((RPT,), jnp.float32)
    zrows = jnp.zeros((RPT, D), jnp.float32)

    deg = _sc_degree(dst_deg, ones_ec, z1)
    deg0 = deg[:NP].reshape(NB, 1, RB)
    deg1 = deg[NP:].reshape(NB, 1, RB)

    u1 = _tc_layer0(xp, W1, deg0, deg1)

    acc1 = _sc_scatter(u1, srcp, dstp, zrows)
    h, u2 = _tc_layer1(acc1, u1, deg0, deg1,
                       b1.reshape(1, D), g1.reshape(1, D), be1.reshape(1, D),
                       xp, W2)

    acc2 = _sc_scatter(u2, srcp, dstp, zrows)
    return _tc_layer2_pool(acc2, u2, deg0, deg1,
                           b2.reshape(1, D), g2.reshape(1, D),
                           be2.reshape(1, D), h, batch_r, Wo, bo)


# speedup vs baseline: 1.0689x; 1.0689x over previous
import functools

import jax
import jax.numpy as jnp
import numpy as np
from jax import lax
from jax.experimental import pallas as pl
from jax.experimental.pallas import tpu as pltpu
from jax.experimental.pallas import tpu_sc as plsc

N = 10000
D = 128
E = 320000
B = 64
EPS = 1e-5

NP = 10240
EP = 327680
NC = 2
NS = 16
NW = NC * NS
ET = EP // NW
EC = 128
CH = ET // EC
NG = 2
GRP = CH // NG
RPT = NP // NS

RB = 1024
NB = NP // RB


@functools.cache
def _build_sc_degree():
    mesh = plsc.VectorSubcoreMesh(core_axis_name="c", subcore_axis_name="s")
    return functools.partial(
        pl.kernel,
        mesh=mesh,
        out_type=jax.ShapeDtypeStruct((NC * NP,), jnp.float32),
        scratch_types=[
            pltpu.VMEM((CH, EC), jnp.int32),
            pltpu.VMEM((EC,), jnp.float32),
            pltpu.VMEM_SHARED((NP,), jnp.float32),
            pltpu.SemaphoreType.DMA,
        ],
    )(_sc_degree_body)


def _sc_degree_body(dst_hbm, ones_hbm, z1_hbm, out_hbm, idx_v, ones_v, deg_sh,
                    sem):
    c = lax.axis_index("c")
    s = lax.axis_index("s")
    wid = c * NS + s
    pltpu.sync_copy(dst_hbm.at[wid], idx_v)
    pltpu.sync_copy(ones_hbm, ones_v)
    pltpu.sync_copy(z1_hbm, deg_sh.at[pl.ds(s * RPT, RPT)])
    plsc.subcore_barrier()

    def fire(j, _):
        pltpu.async_copy(ones_v, deg_sh.at[idx_v.at[j]], sem, add=True)
        return _

    def drain(j, _):
        pltpu.make_async_copy(ones_v, deg_sh.at[idx_v.at[j]], sem).wait()
        return _

    lax.fori_loop(0, CH, fire, None)
    lax.fori_loop(0, CH, drain, None)
    plsc.subcore_barrier()
    pltpu.sync_copy(deg_sh.at[pl.ds(s * RPT, RPT)],
                    out_hbm.at[pl.ds(c * NP + s * RPT, RPT)])


def _sc_degree(dstp, ones128, z1):
    return _build_sc_degree()(dstp, ones128, z1)


@functools.cache
def _build_sc_scatter():
    mesh = plsc.VectorSubcoreMesh(core_axis_name="c", subcore_axis_name="s")
    return functools.partial(
        pl.kernel,
        mesh=mesh,
        out_type=jax.ShapeDtypeStruct((NC * NP, D), jnp.float32),
        scratch_types=[
            pltpu.VMEM((GRP, EC), jnp.int32),
            pltpu.VMEM((GRP, EC), jnp.int32),
            pltpu.VMEM((EC, D), jnp.float32),
            pltpu.VMEM((EC, D), jnp.float32),
            pltpu.VMEM_SHARED((NP, D), jnp.float32),
            pltpu.SemaphoreType.DMA,
            pltpu.SemaphoreType.DMA,
            pltpu.SemaphoreType.DMA,
            pltpu.SemaphoreType.DMA,
        ],
    )(_sc_scatter_body)


def _sc_scatter_body(u_hbm, src_hbm, dst_hbm, zrows_hbm, out_hbm,
                     src_v, dst_v, rows0, rows1, acc_sh,
                     gsem0, gsem1, ssem0, ssem1):
    c = lax.axis_index("c")
    s = lax.axis_index("s")
    wid = c * NS + s
    cz = pltpu.async_copy(zrows_hbm, acc_sh.at[pl.ds(s * RPT, RPT)], ssem0)
    c0 = pltpu.async_copy(src_hbm.at[wid * NG], src_v, gsem0)
    c1 = pltpu.async_copy(dst_hbm.at[wid * NG], dst_v, gsem1)
    c0.wait()
    c1.wait()
    cz.wait()
    plsc.subcore_barrier()

    rows = (rows0, rows1)
    gsem = (gsem0, gsem1)
    ssem = (ssem0, ssem1)

    def body(j, _):
        for b in (0, 1):
            @pl.when(j % 2 == b)
            def _branch():
                nb = 1 - b
                @pl.when(j >= 1)
                def _drain_prev():
                    pltpu.make_async_copy(
                        rows[nb], acc_sh.at[dst_v.at[j - 1]], ssem[nb]).wait()

                @pl.when(j + 1 < GRP)
                def _prefetch():
                    pltpu.async_copy(u_hbm.at[src_v.at[j + 1]], rows[nb],
                                     gsem[nb])

                pltpu.make_async_copy(u_hbm.at[src_v.at[j]], rows[b],
                                      gsem[b]).wait()
                pltpu.async_copy(rows[b], acc_sh.at[dst_v.at[j]], ssem[b],
                                 add=True)
        return _

    for g in range(NG):
        if g > 0:
            cg0 = pltpu.async_copy(src_hbm.at[wid * NG + g], src_v, gsem0)
            cg1 = pltpu.async_copy(dst_hbm.at[wid * NG + g], dst_v, gsem1)
            cg0.wait()
            cg1.wait()
        pltpu.async_copy(u_hbm.at[src_v.at[0]], rows0, gsem0)
        lax.fori_loop(0, GRP, body, None)
        pltpu.make_async_copy(rows[(GRP - 1) % 2], acc_sh.at[dst_v.at[GRP - 1]],
                              ssem[(GRP - 1) % 2]).wait()
    plsc.subcore_barrier()
    pltpu.sync_copy(acc_sh.at[pl.ds(s * RPT, RPT)],
                    out_hbm.at[pl.ds(c * NP + s * RPT, RPT)])


def _sc_scatter(u, srcp, dstp, zrows):
    return _build_sc_scatter()(u, srcp, dstp, zrows)


def _dinv_block(d0_ref, d1_ref):
    return lax.rsqrt(d0_ref[0, 0, :] + d1_ref[0, 0, :] + 1.0)


def _ka_body(x_ref, w_ref, d0_ref, d1_ref, u_ref):
    dinv = _dinv_block(d0_ref, d1_ref)
    hw = jnp.dot(x_ref[...], w_ref[...], preferred_element_type=jnp.float32)
    u_ref[...] = hw * dinv[:, None]


def _tc_layer0(xp, W1, deg0, deg1):
    return pl.pallas_call(
        _ka_body,
        grid=(NB,),
        in_specs=[
            pl.BlockSpec((RB, D), lambda i: (i, 0)),
            pl.BlockSpec((D, D), lambda i: (0, 0)),
            pl.BlockSpec((1, 1, RB), lambda i: (i, 0, 0)),
            pl.BlockSpec((1, 1, RB), lambda i: (i, 0, 0)),
        ],
        out_specs=pl.BlockSpec((RB, D), lambda i: (i, 0)),
        out_shape=jax.ShapeDtypeStruct((NP, D), jnp.float32),
    )(xp, W1, deg0, deg1)


def _layer_post(acc_a, acc_b, u, dinv, b, g, be, res):
    a = (acc_a + acc_b + u) * dinv[:, None] + b[0]
    m = jnp.mean(a, axis=1, keepdims=True)
    v = jnp.mean((a - m) ** 2, axis=1, keepdims=True)
    ln = (a - m) * lax.rsqrt(v + EPS) * g[0] + be[0]
    return jnp.maximum(ln, 0.0) + res


def _kb_body(aa_ref, ab_ref, u1_ref, d0_ref, d1_ref, b1_ref, g1_ref, be1_ref,
             x_ref, w2_ref, h_ref, u2_ref):
    dinv = _dinv_block(d0_ref, d1_ref)
    h = _layer_post(aa_ref[...], ab_ref[...], u1_ref[...], dinv,
                    b1_ref[...], g1_ref[...], be1_ref[...], x_ref[...])
    h_ref[...] = h
    hw2 = jnp.dot(h, w2_ref[...], preferred_element_type=jnp.float32)
    u2_ref[...] = hw2 * dinv[:, None]


def _tc_layer1(acc1, u1, deg0, deg1, b1, g1, be1, xp, W2):
    row = pl.BlockSpec((RB, D), lambda i: (i, 0))
    row_hi = pl.BlockSpec((RB, D), lambda i: (i + NB, 0))
    vec = pl.BlockSpec((1, D), lambda i: (0, 0))
    dspec = pl.BlockSpec((1, 1, RB), lambda i: (i, 0, 0))
    return pl.pallas_call(
        _kb_body,
        grid=(NB,),
        in_specs=[row, row_hi, row, dspec, dspec, vec, vec, vec, row,
                  pl.BlockSpec((D, D), lambda i: (0, 0))],
        out_specs=[row, row],
        out_shape=[jax.ShapeDtypeStruct((NP, D), jnp.float32),
                   jax.ShapeDtypeStruct((NP, D), jnp.float32)],
    )(acc1, acc1, u1, deg0, deg1, b1, g1, be1, xp, W2)


def _kc_body(aa_ref, ab_ref, u2_ref, d0_ref, d1_ref, b2_ref, g2_ref, be2_ref,
             h_ref, ids_ref, wo_ref, bo_ref, out_ref, sums_ref, cnts_ref):
    dinv = _dinv_block(d0_ref, d1_ref)
    t = _layer_post(aa_ref[...], ab_ref[...], u2_ref[...], dinv,
                    b2_ref[...], g2_ref[...], be2_ref[...], h_ref[...])
    ids = ids_ref[0, 0, :]
    onehot = (ids[:, None] == lax.broadcasted_iota(jnp.int32, (RB, B), 1))
    onehot = onehot.astype(jnp.float32)

    @pl.when(pl.program_id(0) == 0)
    def _init():
        sums_ref[...] = jnp.zeros_like(sums_ref)
        cnts_ref[...] = jnp.zeros_like(cnts_ref)

    sums_ref[...] += lax.dot_general(
        onehot, t, (((0,), (0,)), ((), ())),
        preferred_element_type=jnp.float32)
    cnts_ref[...] += jnp.broadcast_to(
        jnp.sum(onehot, axis=0)[:, None], (B, D))

    @pl.when(pl.program_id(0) == NB - 1)
    def _project():
        hg = sums_ref[...] / jnp.maximum(cnts_ref[...], 1.0)
        out_ref[...] = jnp.dot(hg, wo_ref[...],
                               preferred_element_type=jnp.float32) + bo_ref[...]


def _tc_layer2_pool(acc2, u2, deg0, deg1, b2, g2, be2, h, batch_r, Wo, bo):
    row = pl.BlockSpec((RB, D), lambda i: (i, 0))
    row_hi = pl.BlockSpec((RB, D), lambda i: (i + NB, 0))
    vec = pl.BlockSpec((1, D), lambda i: (0, 0))
    dspec = pl.BlockSpec((1, 1, RB), lambda i: (i, 0, 0))
    pool = pl.BlockSpec((B, D), lambda i: (0, 0))
    return pl.pallas_call(
        _kc_body,
        grid=(NB,),
        in_specs=[row, row_hi, row, dspec, dspec, vec, vec, vec, row, dspec,
                  pl.BlockSpec((D, D), lambda i: (0, 0)), vec],
        out_specs=pool,
        out_shape=jax.ShapeDtypeStruct((B, D), jnp.float32),
        scratch_shapes=[pltpu.VMEM((B, D), jnp.float32),
                        pltpu.VMEM((B, D), jnp.float32)],
    )(acc2, acc2, u2, deg0, deg1, b2, g2, be2, h, batch_r, Wo,
      bo.reshape(1, D))


def kernel(x, edge_index, batch, W1, b1, g1, be1, W2, b2, g2, be2, Wo, bo):
    src, dst = edge_index[0], edge_index[1]
    pad_ids = jnp.asarray(N + np.arange(EP - E, dtype=np.int32) % (NP - N))
    srcp = jnp.concatenate([src, pad_ids]).reshape(NW * NG, GRP, EC)
    dstp = jnp.concatenate([dst, pad_ids]).reshape(NW * NG, GRP, EC)
    dst_deg = dstp.reshape(NW, CH, EC)
    xp = jnp.pad(x, ((0, NP - N), (0, 0)))
    batch_r = jnp.pad(batch, (0, NP - N), constant_values=B).reshape(NB, 1, RB)

    ones_ec = jnp.ones((EC,), jnp.float32)
    z1 = jnp.zeros((RPT,), jnp.float32)
    zrows = jnp.zeros((RPT, D), jnp.float32)

    deg = _sc_degree(dst_deg, ones_ec, z1)
    deg0 = deg[:NP].reshape(NB, 1, RB)
    deg1 = deg[NP:].reshape(NB, 1, RB)

    u1 = _tc_layer0(xp, W1, deg0, deg1)

    acc1 = _sc_scatter(u1, srcp, dstp, zrows)
    h, u2 = _tc_layer1(acc1, u1, deg0, deg1,
                       b1.reshape(1, D), g1.reshape(1, D), be1.reshape(1, D),
                       xp, W2)

    acc2 = _sc_scatter(u2, srcp, dstp, zrows)
    return _tc_layer2_pool(acc2, u2, deg0, deg1,
                           b2.reshape(1, D), g2.reshape(1, D),
                           be2.reshape(1, D), h, batch_r, Wo, bo)


# speedup vs baseline: 1.0864x; 1.0164x over previous
import functools

import jax
import jax.numpy as jnp
import numpy as np
from jax import lax
from jax.experimental import pallas as pl
from jax.experimental.pallas import tpu as pltpu
from jax.experimental.pallas import tpu_sc as plsc

N = 10000
D = 128
E = 320000
B = 64
EPS = 1e-5

NP = 10240
EP = 327680
NC = 2
NS = 16
NW = NC * NS
ET = EP // NW
EC = 128
CH = ET // EC
NG = 2
GRP = CH // NG
RPT = NP // NS

RB = 2048
NB = NP // RB


@functools.cache
def _build_sc_degree():
    mesh = plsc.VectorSubcoreMesh(core_axis_name="c", subcore_axis_name="s")
    return functools.partial(
        pl.kernel,
        mesh=mesh,
        out_type=jax.ShapeDtypeStruct((NC * NP,), jnp.float32),
        scratch_types=[
            pltpu.VMEM((CH, EC), jnp.int32),
            pltpu.VMEM((EC,), jnp.float32),
            pltpu.VMEM_SHARED((NP,), jnp.float32),
            pltpu.SemaphoreType.DMA,
        ],
    )(_sc_degree_body)


def _sc_degree_body(dst_hbm, ones_hbm, z1_hbm, out_hbm, idx_v, ones_v, deg_sh,
                    sem):
    c = lax.axis_index("c")
    s = lax.axis_index("s")
    wid = c * NS + s
    pltpu.sync_copy(dst_hbm.at[wid], idx_v)
    pltpu.sync_copy(ones_hbm, ones_v)
    pltpu.sync_copy(z1_hbm, deg_sh.at[pl.ds(s * RPT, RPT)])
    plsc.subcore_barrier()

    def fire(j, _):
        pltpu.async_copy(ones_v, deg_sh.at[idx_v.at[j]], sem, add=True)
        return _

    def drain(j, _):
        pltpu.make_async_copy(ones_v, deg_sh.at[idx_v.at[j]], sem).wait()
        return _

    lax.fori_loop(0, CH, fire, None)
    lax.fori_loop(0, CH, drain, None)
    plsc.subcore_barrier()
    pltpu.sync_copy(deg_sh.at[pl.ds(s * RPT, RPT)],
                    out_hbm.at[pl.ds(c * NP + s * RPT, RPT)])


def _sc_degree(dstp, ones128, z1):
    return _build_sc_degree()(dstp, ones128, z1)


@functools.cache
def _build_sc_scatter():
    mesh = plsc.VectorSubcoreMesh(core_axis_name="c", subcore_axis_name="s")
    return functools.partial(
        pl.kernel,
        mesh=mesh,
        out_type=jax.ShapeDtypeStruct((NC * NP, D), jnp.float32),
        scratch_types=[
            pltpu.VMEM((GRP, EC), jnp.int32),
            pltpu.VMEM((GRP, EC), jnp.int32),
            pltpu.VMEM((EC, D), jnp.float32),
            pltpu.VMEM((EC, D), jnp.float32),
            pltpu.VMEM_SHARED((NP, D), jnp.float32),
            pltpu.SemaphoreType.DMA,
            pltpu.SemaphoreType.DMA,
            pltpu.SemaphoreType.DMA,
            pltpu.SemaphoreType.DMA,
        ],
    )(_sc_scatter_body)


def _sc_scatter_body(u_hbm, src_hbm, dst_hbm, zrows_hbm, out_hbm,
                     src_v, dst_v, rows0, rows1, acc_sh,
                     gsem0, gsem1, ssem0, ssem1):
    c = lax.axis_index("c")
    s = lax.axis_index("s")
    wid = c * NS + s
    cz = pltpu.async_copy(zrows_hbm, acc_sh.at[pl.ds(s * RPT, RPT)], ssem0)
    c0 = pltpu.async_copy(src_hbm.at[wid * NG], src_v, gsem0)
    c1 = pltpu.async_copy(dst_hbm.at[wid * NG], dst_v, gsem1)
    c0.wait()
    c1.wait()
    cz.wait()
    plsc.subcore_barrier()

    rows = (rows0, rows1)
    gsem = (gsem0, gsem1)
    ssem = (ssem0, ssem1)

    def body(j, _):
        for b in (0, 1):
            @pl.when(j % 2 == b)
            def _branch():
                nb = 1 - b
                @pl.when(j >= 1)
                def _drain_prev():
                    pltpu.make_async_copy(
                        rows[nb], acc_sh.at[dst_v.at[j - 1]], ssem[nb]).wait()

                @pl.when(j + 1 < GRP)
                def _prefetch():
                    pltpu.async_copy(u_hbm.at[src_v.at[j + 1]], rows[nb],
                                     gsem[nb])

                pltpu.make_async_copy(u_hbm.at[src_v.at[j]], rows[b],
                                      gsem[b]).wait()
                pltpu.async_copy(rows[b], acc_sh.at[dst_v.at[j]], ssem[b],
                                 add=True)
        return _

    for g in range(NG):
        if g > 0:
            cg0 = pltpu.async_copy(src_hbm.at[wid * NG + g], src_v, gsem0)
            cg1 = pltpu.async_copy(dst_hbm.at[wid * NG + g], dst_v, gsem1)
            cg0.wait()
            cg1.wait()
        pltpu.async_copy(u_hbm.at[src_v.at[0]], rows0, gsem0)
        lax.fori_loop(0, GRP, body, None)
        pltpu.make_async_copy(rows[(GRP - 1) % 2], acc_sh.at[dst_v.at[GRP - 1]],
                              ssem[(GRP - 1) % 2]).wait()
    plsc.subcore_barrier()
    pltpu.sync_copy(acc_sh.at[pl.ds(s * RPT, RPT)],
                    out_hbm.at[pl.ds(c * NP + s * RPT, RPT)])


def _sc_scatter(u, srcp, dstp, zrows):
    return _build_sc_scatter()(u, srcp, dstp, zrows)


def _dinv_block(d0_ref, d1_ref):
    return lax.rsqrt(d0_ref[0, 0, :] + d1_ref[0, 0, :] + 1.0)


def _ka_body(x_ref, w_ref, d0_ref, d1_ref, u_ref):
    dinv = _dinv_block(d0_ref, d1_ref)
    hw = jnp.dot(x_ref[...], w_ref[...], preferred_element_type=jnp.float32)
    u_ref[...] = hw * dinv[:, None]


def _tc_layer0(xp, W1, deg0, deg1):
    return pl.pallas_call(
        _ka_body,
        grid=(NB,),
        in_specs=[
            pl.BlockSpec((RB, D), lambda i: (i, 0)),
            pl.BlockSpec((D, D), lambda i: (0, 0)),
            pl.BlockSpec((1, 1, RB), lambda i: (i, 0, 0)),
            pl.BlockSpec((1, 1, RB), lambda i: (i, 0, 0)),
        ],
        out_specs=pl.BlockSpec((RB, D), lambda i: (i, 0)),
        out_shape=jax.ShapeDtypeStruct((NP, D), jnp.float32),
    )(xp, W1, deg0, deg1)


def _layer_post(acc_a, acc_b, u, dinv, b, g, be, res):
    a = (acc_a + acc_b + u) * dinv[:, None] + b[0]
    m = jnp.mean(a, axis=1, keepdims=True)
    v = jnp.mean((a - m) ** 2, axis=1, keepdims=True)
    ln = (a - m) * lax.rsqrt(v + EPS) * g[0] + be[0]
    return jnp.maximum(ln, 0.0) + res


def _kb_body(aa_ref, ab_ref, u1_ref, d0_ref, d1_ref, b1_ref, g1_ref, be1_ref,
             x_ref, w2_ref, h_ref, u2_ref):
    dinv = _dinv_block(d0_ref, d1_ref)
    h = _layer_post(aa_ref[...], ab_ref[...], u1_ref[...], dinv,
                    b1_ref[...], g1_ref[...], be1_ref[...], x_ref[...])
    h_ref[...] = h
    hw2 = jnp.dot(h, w2_ref[...], preferred_element_type=jnp.float32)
    u2_ref[...] = hw2 * dinv[:, None]


def _tc_layer1(acc1, u1, deg0, deg1, b1, g1, be1, xp, W2):
    row = pl.BlockSpec((RB, D), lambda i: (i, 0))
    row_hi = pl.BlockSpec((RB, D), lambda i: (i + NB, 0))
    vec = pl.BlockSpec((1, D), lambda i: (0, 0))
    dspec = pl.BlockSpec((1, 1, RB), lambda i: (i, 0, 0))
    return pl.pallas_call(
        _kb_body,
        grid=(NB,),
        in_specs=[row, row_hi, row, dspec, dspec, vec, vec, vec, row,
                  pl.BlockSpec((D, D), lambda i: (0, 0))],
        out_specs=[row, row],
        out_shape=[jax.ShapeDtypeStruct((NP, D), jnp.float32),
                   jax.ShapeDtypeStruct((NP, D), jnp.float32)],
    )(acc1, acc1, u1, deg0, deg1, b1, g1, be1, xp, W2)


def _kc_body(aa_ref, ab_ref, u2_ref, d0_ref, d1_ref, b2_ref, g2_ref, be2_ref,
             h_ref, ids_ref, wo_ref, bo_ref, out_ref, sums_ref, cnts_ref):
    dinv = _dinv_block(d0_ref, d1_ref)
    t = _layer_post(aa_ref[...], ab_ref[...], u2_ref[...], dinv,
                    b2_ref[...], g2_ref[...], be2_ref[...], h_ref[...])
    ids = ids_ref[0, 0, :]
    onehot = (ids[:, None] == lax.broadcasted_iota(jnp.int32, (RB, B), 1))
    onehot = onehot.astype(jnp.float32)

    @pl.when(pl.program_id(0) == 0)
    def _init():
        sums_ref[...] = jnp.zeros_like(sums_ref)
        cnts_ref[...] = jnp.zeros_like(cnts_ref)

    sums_ref[...] += lax.dot_general(
        onehot, t, (((0,), (0,)), ((), ())),
        preferred_element_type=jnp.float32)
    cnts_ref[...] += jnp.broadcast_to(
        jnp.sum(onehot, axis=0)[:, None], (B, D))

    @pl.when(pl.program_id(0) == NB - 1)
    def _project():
        hg = sums_ref[...] / jnp.maximum(cnts_ref[...], 1.0)
        out_ref[...] = jnp.dot(hg, wo_ref[...],
                               preferred_element_type=jnp.float32) + bo_ref[...]


def _tc_layer2_pool(acc2, u2, deg0, deg1, b2, g2, be2, h, batch_r, Wo, bo):
    row = pl.BlockSpec((RB, D), lambda i: (i, 0))
    row_hi = pl.BlockSpec((RB, D), lambda i: (i + NB, 0))
    vec = pl.BlockSpec((1, D), lambda i: (0, 0))
    dspec = pl.BlockSpec((1, 1, RB), lambda i: (i, 0, 0))
    pool = pl.BlockSpec((B, D), lambda i: (0, 0))
    return pl.pallas_call(
        _kc_body,
        grid=(NB,),
        in_specs=[row, row_hi, row, dspec, dspec, vec, vec, vec, row, dspec,
                  pl.BlockSpec((D, D), lambda i: (0, 0)), vec],
        out_specs=pool,
        out_shape=jax.ShapeDtypeStruct((B, D), jnp.float32),
        scratch_shapes=[pltpu.VMEM((B, D), jnp.float32),
                        pltpu.VMEM((B, D), jnp.float32)],
    )(acc2, acc2, u2, deg0, deg1, b2, g2, be2, h, batch_r, Wo,
      bo.reshape(1, D))


def kernel(x, edge_index, batch, W1, b1, g1, be1, W2, b2, g2, be2, Wo, bo):
    src, dst = edge_index[0], edge_index[1]
    pad_ids = jnp.asarray(N + np.arange(EP - E, dtype=np.int32) % (NP - N))
    srcp = jnp.concatenate([src, pad_ids]).reshape(NW * NG, GRP, EC)
    dstp = jnp.concatenate([dst, pad_ids]).reshape(NW * NG, GRP, EC)
    dst_deg = dstp.reshape(NW, CH, EC)
    xp = jnp.pad(x, ((0, NP - N), (0, 0)))
    batch_r = jnp.pad(batch, (0, NP - N), constant_values=B).reshape(NB, 1, RB)

    ones_ec = jnp.ones((EC,), jnp.float32)
    z1 = jnp.zeros((RPT,), jnp.float32)
    zrows = jnp.zeros((RPT, D), jnp.float32)

    deg = _sc_degree(dst_deg, ones_ec, z1)
    deg0 = deg[:NP].reshape(NB, 1, RB)
    deg1 = deg[NP:].reshape(NB, 1, RB)

    u1 = _tc_layer0(xp, W1, deg0, deg1)

    acc1 = _sc_scatter(u1, srcp, dstp, zrows)
    h, u2 = _tc_layer1(acc1, u1, deg0, deg1,
                       b1.reshape(1, D), g1.reshape(1, D), be1.reshape(1, D),
                       xp, W2)

    acc2 = _sc_scatter(u2, srcp, dstp, zrows)
    return _tc_layer2_pool(acc2, u2, deg0, deg1,
                           b2.reshape(1, D), g2.reshape(1, D),
                           be2.reshape(1, D), h, batch_r, Wo, bo)


# speedup vs baseline: 1.0933x; 1.0063x over previous
import functools

import jax
import jax.numpy as jnp
import numpy as np
from jax import lax
from jax.experimental import pallas as pl
from jax.experimental.pallas import tpu as pltpu
from jax.experimental.pallas import tpu_sc as plsc

N = 10000
D = 128
E = 320000
B = 64
EPS = 1e-5

NP = 10240
EP = 327680
NC = 2
NS = 16
NW = NC * NS
ET = EP // NW
EC = 128
CH = ET // EC
NG = 2
GRP = CH // NG
RPT = NP // NS

RB = 5120
NB = NP // RB


@functools.cache
def _build_sc_degree():
    mesh = plsc.VectorSubcoreMesh(core_axis_name="c", subcore_axis_name="s")
    return functools.partial(
        pl.kernel,
        mesh=mesh,
        out_type=jax.ShapeDtypeStruct((NC * NP,), jnp.float32),
        scratch_types=[
            pltpu.VMEM((CH, EC), jnp.int32),
            pltpu.VMEM((EC,), jnp.float32),
            pltpu.VMEM_SHARED((NP,), jnp.float32),
            pltpu.SemaphoreType.DMA,
        ],
    )(_sc_degree_body)


def _sc_degree_body(dst_hbm, ones_hbm, z1_hbm, out_hbm, idx_v, ones_v, deg_sh,
                    sem):
    c = lax.axis_index("c")
    s = lax.axis_index("s")
    wid = c * NS + s
    pltpu.sync_copy(dst_hbm.at[wid], idx_v)
    pltpu.sync_copy(ones_hbm, ones_v)
    pltpu.sync_copy(z1_hbm, deg_sh.at[pl.ds(s * RPT, RPT)])
    plsc.subcore_barrier()

    def fire(j, _):
        pltpu.async_copy(ones_v, deg_sh.at[idx_v.at[j]], sem, add=True)
        return _

    def drain(j, _):
        pltpu.make_async_copy(ones_v, deg_sh.at[idx_v.at[j]], sem).wait()
        return _

    lax.fori_loop(0, CH, fire, None)
    lax.fori_loop(0, CH, drain, None)
    plsc.subcore_barrier()
    pltpu.sync_copy(deg_sh.at[pl.ds(s * RPT, RPT)],
                    out_hbm.at[pl.ds(c * NP + s * RPT, RPT)])


def _sc_degree(dstp, ones128, z1):
    return _build_sc_degree()(dstp, ones128, z1)


@functools.cache
def _build_sc_scatter():
    mesh = plsc.VectorSubcoreMesh(core_axis_name="c", subcore_axis_name="s")
    return functools.partial(
        pl.kernel,
        mesh=mesh,
        out_type=jax.ShapeDtypeStruct((NC * NP, D), jnp.float32),
        scratch_types=[
            pltpu.VMEM((GRP, EC), jnp.int32),
            pltpu.VMEM((GRP, EC), jnp.int32),
            pltpu.VMEM((EC, D), jnp.float32),
            pltpu.VMEM((EC, D), jnp.float32),
            pltpu.VMEM_SHARED((NP, D), jnp.float32),
            pltpu.SemaphoreType.DMA,
            pltpu.SemaphoreType.DMA,
            pltpu.SemaphoreType.DMA,
            pltpu.SemaphoreType.DMA,
        ],
    )(_sc_scatter_body)


def _sc_scatter_body(u_hbm, src_hbm, dst_hbm, zrows_hbm, out_hbm,
                     src_v, dst_v, rows0, rows1, acc_sh,
                     gsem0, gsem1, ssem0, ssem1):
    c = lax.axis_index("c")
    s = lax.axis_index("s")
    wid = c * NS + s
    cz = pltpu.async_copy(zrows_hbm, acc_sh.at[pl.ds(s * RPT, RPT)], ssem0)
    c0 = pltpu.async_copy(src_hbm.at[wid * NG], src_v, gsem0)
    c1 = pltpu.async_copy(dst_hbm.at[wid * NG], dst_v, gsem1)
    c0.wait()
    c1.wait()
    cz.wait()
    plsc.subcore_barrier()

    rows = (rows0, rows1)
    gsem = (gsem0, gsem1)
    ssem = (ssem0, ssem1)

    def body(j, _):
        for b in (0, 1):
            @pl.when(j % 2 == b)
            def _branch():
                nb = 1 - b
                @pl.when(j >= 1)
                def _drain_prev():
                    pltpu.make_async_copy(
                        rows[nb], acc_sh.at[dst_v.at[j - 1]], ssem[nb]).wait()

                @pl.when(j + 1 < GRP)
                def _prefetch():
                    pltpu.async_copy(u_hbm.at[src_v.at[j + 1]], rows[nb],
                                     gsem[nb])

                pltpu.make_async_copy(u_hbm.at[src_v.at[j]], rows[b],
                                      gsem[b]).wait()
                pltpu.async_copy(rows[b], acc_sh.at[dst_v.at[j]], ssem[b],
                                 add=True)
        return _

    for g in range(NG):
        if g > 0:
            cg0 = pltpu.async_copy(src_hbm.at[wid * NG + g], src_v, gsem0)
            cg1 = pltpu.async_copy(dst_hbm.at[wid * NG + g], dst_v, gsem1)
            cg0.wait()
            cg1.wait()
        pltpu.async_copy(u_hbm.at[src_v.at[0]], rows0, gsem0)
        lax.fori_loop(0, GRP, body, None)
        pltpu.make_async_copy(rows[(GRP - 1) % 2], acc_sh.at[dst_v.at[GRP - 1]],
                              ssem[(GRP - 1) % 2]).wait()
    plsc.subcore_barrier()
    pltpu.sync_copy(acc_sh.at[pl.ds(s * RPT, RPT)],
                    out_hbm.at[pl.ds(c * NP + s * RPT, RPT)])


def _sc_scatter(u, srcp, dstp, zrows):
    return _build_sc_scatter()(u, srcp, dstp, zrows)


def _dinv_block(d0_ref, d1_ref):
    return lax.rsqrt(d0_ref[0, 0, :] + d1_ref[0, 0, :] + 1.0)


def _ka_body(x_ref, w_ref, d0_ref, d1_ref, u_ref):
    dinv = _dinv_block(d0_ref, d1_ref)
    hw = jnp.dot(x_ref[...], w_ref[...], preferred_element_type=jnp.float32)
    u_ref[...] = hw * dinv[:, None]


def _tc_layer0(xp, W1, deg0, deg1):
    return pl.pallas_call(
        _ka_body,
        grid=(NB,),
        in_specs=[
            pl.BlockSpec((RB, D), lambda i: (i, 0)),
            pl.BlockSpec((D, D), lambda i: (0, 0)),
            pl.BlockSpec((1, 1, RB), lambda i: (i, 0, 0)),
            pl.BlockSpec((1, 1, RB), lambda i: (i, 0, 0)),
        ],
        out_specs=pl.BlockSpec((RB, D), lambda i: (i, 0)),
        out_shape=jax.ShapeDtypeStruct((NP, D), jnp.float32),
    )(xp, W1, deg0, deg1)


def _layer_post(acc_a, acc_b, u, dinv, b, g, be, res):
    a = (acc_a + acc_b + u) * dinv[:, None] + b[0]
    m = jnp.mean(a, axis=1, keepdims=True)
    v = jnp.mean((a - m) ** 2, axis=1, keepdims=True)
    ln = (a - m) * lax.rsqrt(v + EPS) * g[0] + be[0]
    return jnp.maximum(ln, 0.0) + res


def _kb_body(aa_ref, ab_ref, u1_ref, d0_ref, d1_ref, b1_ref, g1_ref, be1_ref,
             x_ref, w2_ref, h_ref, u2_ref):
    dinv = _dinv_block(d0_ref, d1_ref)
    h = _layer_post(aa_ref[...], ab_ref[...], u1_ref[...], dinv,
                    b1_ref[...], g1_ref[...], be1_ref[...], x_ref[...])
    h_ref[...] = h
    hw2 = jnp.dot(h, w2_ref[...], preferred_element_type=jnp.float32)
    u2_ref[...] = hw2 * dinv[:, None]


def _tc_layer1(acc1, u1, deg0, deg1, b1, g1, be1, xp, W2):
    row = pl.BlockSpec((RB, D), lambda i: (i, 0))
    row_hi = pl.BlockSpec((RB, D), lambda i: (i + NB, 0))
    vec = pl.BlockSpec((1, D), lambda i: (0, 0))
    dspec = pl.BlockSpec((1, 1, RB), lambda i: (i, 0, 0))
    return pl.pallas_call(
        _kb_body,
        grid=(NB,),
        in_specs=[row, row_hi, row, dspec, dspec, vec, vec, vec, row,
                  pl.BlockSpec((D, D), lambda i: (0, 0))],
        out_specs=[row, row],
        out_shape=[jax.ShapeDtypeStruct((NP, D), jnp.float32),
                   jax.ShapeDtypeStruct((NP, D), jnp.float32)],
    )(acc1, acc1, u1, deg0, deg1, b1, g1, be1, xp, W2)


def _kc_body(aa_ref, ab_ref, u2_ref, d0_ref, d1_ref, b2_ref, g2_ref, be2_ref,
             h_ref, ids_ref, wo_ref, bo_ref, out_ref, sums_ref, cnts_ref):
    dinv = _dinv_block(d0_ref, d1_ref)
    t = _layer_post(aa_ref[...], ab_ref[...], u2_ref[...], dinv,
                    b2_ref[...], g2_ref[...], be2_ref[...], h_ref[...])
    ids = ids_ref[0, 0, :]
    onehot = (ids[:, None] == lax.broadcasted_iota(jnp.int32, (RB, B), 1))
    onehot = onehot.astype(jnp.float32)

    @pl.when(pl.program_id(0) == 0)
    def _init():
        sums_ref[...] = jnp.zeros_like(sums_ref)
        cnts_ref[...] = jnp.zeros_like(cnts_ref)

    sums_ref[...] += lax.dot_general(
        onehot, t, (((0,), (0,)), ((), ())),
        preferred_element_type=jnp.float32)
    cnts_ref[...] += jnp.broadcast_to(
        jnp.sum(onehot, axis=0)[:, None], (B, D))

    @pl.when(pl.program_id(0) == NB - 1)
    def _project():
        hg = sums_ref[...] / jnp.maximum(cnts_ref[...], 1.0)
        out_ref[...] = jnp.dot(hg, wo_ref[...],
                               preferred_element_type=jnp.float32) + bo_ref[...]


def _tc_layer2_pool(acc2, u2, deg0, deg1, b2, g2, be2, h, batch_r, Wo, bo):
    row = pl.BlockSpec((RB, D), lambda i: (i, 0))
    row_hi = pl.BlockSpec((RB, D), lambda i: (i + NB, 0))
    vec = pl.BlockSpec((1, D), lambda i: (0, 0))
    dspec = pl.BlockSpec((1, 1, RB), lambda i: (i, 0, 0))
    pool = pl.BlockSpec((B, D), lambda i: (0, 0))
    return pl.pallas_call(
        _kc_body,
        grid=(NB,),
        in_specs=[row, row_hi, row, dspec, dspec, vec, vec, vec, row, dspec,
                  pl.BlockSpec((D, D), lambda i: (0, 0)), vec],
        out_specs=pool,
        out_shape=jax.ShapeDtypeStruct((B, D), jnp.float32),
        scratch_shapes=[pltpu.VMEM((B, D), jnp.float32),
                        pltpu.VMEM((B, D), jnp.float32)],
    )(acc2, acc2, u2, deg0, deg1, b2, g2, be2, h, batch_r, Wo,
      bo.reshape(1, D))


def kernel(x, edge_index, batch, W1, b1, g1, be1, W2, b2, g2, be2, Wo, bo):
    src, dst = edge_index[0], edge_index[1]
    pad_ids = jnp.asarray(N + np.arange(EP - E, dtype=np.int32) % (NP - N))
    srcp = jnp.concatenate([src, pad_ids]).reshape(NW * NG, GRP, EC)
    dstp = jnp.concatenate([dst, pad_ids]).reshape(NW * NG, GRP, EC)
    dst_deg = dstp.reshape(NW, CH, EC)
    xp = jnp.pad(x, ((0, NP - N), (0, 0)))
    batch_r = jnp.pad(batch, (0, NP - N), constant_values=B).reshape(NB, 1, RB)

    ones_ec = jnp.ones((EC,), jnp.float32)
    z1 = jnp.zeros((RPT,), jnp.float32)
    zrows = jnp.zeros((RPT, D), jnp.float32)

    deg = _sc_degree(dst_deg, ones_ec, z1)
    deg0 = deg[:NP].reshape(NB, 1, RB)
    deg1 = deg[NP:].reshape(NB, 1, RB)

    u1 = _tc_layer0(xp, W1, deg0, deg1)

    acc1 = _sc_scatter(u1, srcp, dstp, zrows)
    h, u2 = _tc_layer1(acc1, u1, deg0, deg1,
                       b1.reshape(1, D), g1.reshape(1, D), be1.reshape(1, D),
                       xp, W2)

    acc2 = _sc_scatter(u2, srcp, dstp, zrows)
    return _tc_layer2_pool(acc2, u2, deg0, deg1,
                           b2.reshape(1, D), g2.reshape(1, D),
                           be2.reshape(1, D), h, batch_r, Wo, bo)
